# baseline (speedup 1.0000x reference)
import functools

import jax
import jax.numpy as jnp
from jax import lax
from jax.experimental import pallas as pl
from jax.experimental.pallas import tpu as pltpu
from jax.experimental.pallas import tpu_sc as plsc

_COMMITMENT_COST = 0.25
_E = 8192
_D = 256
_TT = 1024
_CH = 256
_BIGF = 3.0e10

_SC_CORES = 2
_SC_SUBCORES = 16
_SC_WORKERS = _SC_CORES * _SC_SUBCORES


def _argmin_body(z_ref, cb_ref, idx_ref, loss_ref, cbsq_ref, iotaf_ref,
                 acc_ref, *, n_tiles):
    i = pl.program_id(0)
    zb = z_ref[0]
    zf = lax.transpose(zb, (1, 0))
    cb = cb_ref[...]

    @pl.when(i == 0)
    def _prep():
        ones = jnp.ones((8, _D), dtype=jnp.float32)
        cbsq_ref[...] = lax.dot_general(
            ones, cb * cb, (((1,), (1,)), ((), ())),
            preferred_element_type=jnp.float32)
        iotaf_ref[...] = lax.broadcasted_iota(
            jnp.int32, (8, _E), 1).astype(jnp.float32)

    ii = iotaf_ref[0:1, :]
    cbsq = cbsq_ref[0:1, :]
    tile_loss = None
    for h in range(_TT // _CH):
        zfh = zf[h * _CH:(h + 1) * _CH, :]
        s2 = lax.dot_general(zfh + zfh, cb, (((1,), (1,)), ((), ())),
                             preferred_element_type=jnp.float32)
        z_sq = jnp.sum(zfh * zfh, axis=1, keepdims=True)
        d = (z_sq - s2) + cbsq
        minv = jnp.min(d, axis=1, keepdims=True)
        idxf = jnp.min(jnp.where(d == minv, ii, _BIGF), axis=1)
        idx_ref[0, 0, h * _CH:(h + 1) * _CH] = idxf.astype(jnp.int32)
        part = jnp.sum(minv)
        tile_loss = part if tile_loss is None else tile_loss + part

    @pl.when(i == 0)
    def _init():
        acc_ref[0, 0] = tile_loss

    @pl.when(i != 0)
    def _acc():
        acc_ref[0, 0] = acc_ref[0, 0] + tile_loss

    @pl.when(i == n_tiles - 1)
    def _fin():
        loss_ref[0, 0] = acc_ref[0, 0]


def _argmin_call(z4, codebook):
    B, _, HW = z4.shape
    n_tiles = (B * HW) // _TT
    per_b = HW // _TT
    return pl.pallas_call(
        functools.partial(_argmin_body, n_tiles=n_tiles),
        grid=(n_tiles,),
        in_specs=[
            pl.BlockSpec((1, _D, _TT), lambda i, pb=per_b: (i // pb, 0, i % pb)),
            pl.BlockSpec((_E, _D), lambda i: (0, 0)),
        ],
        out_specs=[
            pl.BlockSpec((1, 1, _TT), lambda i: (i, 0, 0)),
            pl.BlockSpec(memory_space=pltpu.SMEM),
        ],
        out_shape=[
            jax.ShapeDtypeStruct((n_tiles, 1, _TT), jnp.int32),
            jax.ShapeDtypeStruct((1, 1), jnp.float32),
        ],
        scratch_shapes=[pltpu.VMEM((8, _E), jnp.float32),
                        pltpu.VMEM((8, _E), jnp.float32),
                        pltpu.SMEM((1, 1), jnp.float32)],
    )(z4, codebook)


def _sc_gather(codebook, idx_flat, n_tokens):
    b_per_w = n_tokens // _SC_WORKERS
    mesh = plsc.VectorSubcoreMesh(core_axis_name="c", subcore_axis_name="s")

    @functools.partial(
        pl.kernel,
        mesh=mesh,
        out_type=jax.ShapeDtypeStruct((n_tokens, _D), jnp.float32),
        scratch_types=[
            pltpu.VMEM((b_per_w,), jnp.int32),
            pltpu.VMEM((b_per_w, _D), jnp.float32),
            pltpu.SemaphoreType.DMA,
        ],
    )
    def gather_k(cb_hbm, idx_hbm, out_hbm, idx_v, rows_v, sem):
        wid = lax.axis_index("s") * _SC_CORES + lax.axis_index("c")
        base = wid * b_per_w
        pltpu.sync_copy(idx_hbm.at[pl.ds(base, b_per_w)], idx_v)
        pltpu.async_copy(cb_hbm.at[idx_v], rows_v, sem).wait()
        pltpu.sync_copy(rows_v, out_hbm.at[pl.ds(base, b_per_w)])

    return gather_k(codebook, idx_flat)


def kernel(z, codebook):
    B, C, H, W = z.shape
    z4 = z.reshape(B, C, H * W)
    halves = []
    for lo, hi in ((0, B // 2), (B // 2, B)):
        nb = hi - lo
        nt = nb * H * W
        idx3, lsum = _argmin_call(z4[lo:hi], codebook)
        halves.append((lo, hi, nb, nt, idx3, lsum))
    outs = []
    for lo, hi, nb, nt, idx3, lsum in halves:
        qf = _sc_gather(codebook, idx3.reshape(nt), nt)
        quantized = jnp.transpose(qf.reshape(nb, H, W, C), (0, 3, 1, 2))
        zh = z[lo:hi]
        outs.append((zh + (quantized - zh), idx3.reshape(nb, H, W)))
    quantized_st = jnp.concatenate([o[0] for o in outs], axis=0)
    idx_out = jnp.concatenate([o[1] for o in outs], axis=0)
    total = halves[0][5][0, 0] + halves[1][5][0, 0]
    codebook_loss = total / jnp.float32(B * C * H * W)
    commitment_loss = _COMMITMENT_COST * codebook_loss
    return (quantized_st, idx_out, commitment_loss, codebook_loss)


# speedup vs baseline: 1.1379x; 1.1379x over previous
import functools

import jax
import jax.numpy as jnp
from jax import lax
from jax.experimental import pallas as pl
from jax.experimental.pallas import tpu as pltpu
from jax.experimental.pallas import tpu_sc as plsc

_COMMITMENT_COST = 0.25
_E = 8192
_D = 256
_TT = 1024
_CH = 256
_BIGF = 3.0e10

_SC_CORES = 2
_SC_SUBCORES = 16
_SC_WORKERS = _SC_CORES * _SC_SUBCORES


def _argmin_body(z_ref, cb_ref, idx_ref, loss_ref, cbsq_ref, iotaf_ref,
                 acc_ref, *, n_tiles, n_elems):
    i = pl.program_id(0)
    zb = z_ref[0]
    zf = lax.transpose(zb, (1, 0))
    cb = cb_ref[...]

    @pl.when(i == 0)
    def _prep():
        ones = jnp.ones((8, _D), dtype=jnp.float32)
        cbsq_ref[...] = lax.dot_general(
            ones, cb * cb, (((1,), (1,)), ((), ())),
            preferred_element_type=jnp.float32)
        iotaf_ref[...] = lax.broadcasted_iota(
            jnp.int32, (8, _E), 1).astype(jnp.float32)

    ii = iotaf_ref[0:1, :]
    cbsq = cbsq_ref[0:1, :]
    tile_loss = None
    for h in range(_TT // _CH):
        zfh = zf[h * _CH:(h + 1) * _CH, :]
        s2 = lax.dot_general(zfh + zfh, cb, (((1,), (1,)), ((), ())),
                             preferred_element_type=jnp.float32)
        z_sq = jnp.sum(zfh * zfh, axis=1, keepdims=True)
        d = (z_sq - s2) + cbsq
        minv = jnp.min(d, axis=1, keepdims=True)
        idxf = jnp.min(jnp.where(d == minv, ii, _BIGF), axis=1)
        idx_ref[0, 0, h * _CH:(h + 1) * _CH] = idxf.astype(jnp.int32)
        part = jnp.sum(minv)
        tile_loss = part if tile_loss is None else tile_loss + part

    @pl.when(i == 0)
    def _init():
        acc_ref[0, 0] = tile_loss

    @pl.when(i != 0)
    def _acc():
        acc_ref[0, 0] = acc_ref[0, 0] + tile_loss

    @pl.when(i == n_tiles - 1)
    def _fin():
        cl = acc_ref[0, 0] / jnp.float32(n_elems)
        loss_ref[0, 0] = cl
        loss_ref[0, 1] = _COMMITMENT_COST * cl


def _argmin_call(z4, codebook):
    B, _, HW = z4.shape
    n_tiles = (B * HW) // _TT
    per_b = HW // _TT
    return pl.pallas_call(
        functools.partial(_argmin_body, n_tiles=n_tiles,
                          n_elems=B * HW * _D),
        grid=(n_tiles,),
        in_specs=[
            pl.BlockSpec((1, _D, _TT), lambda i, pb=per_b: (i // pb, 0, i % pb)),
            pl.BlockSpec((_E, _D), lambda i: (0, 0)),
        ],
        out_specs=[
            pl.BlockSpec((1, 1, _TT), lambda i: (i, 0, 0)),
            pl.BlockSpec(memory_space=pltpu.SMEM),
        ],
        out_shape=[
            jax.ShapeDtypeStruct((n_tiles, 1, _TT), jnp.int32),
            jax.ShapeDtypeStruct((1, 2), jnp.float32),
        ],
        scratch_shapes=[pltpu.VMEM((8, _E), jnp.float32),
                        pltpu.VMEM((8, _E), jnp.float32),
                        pltpu.SMEM((1, 1), jnp.float32)],
    )(z4, codebook)


def _sc_gather(codebook, idx_flat, n_tokens):
    b_per_w = n_tokens // _SC_WORKERS
    mesh = plsc.VectorSubcoreMesh(core_axis_name="c", subcore_axis_name="s")

    @functools.partial(
        pl.kernel,
        mesh=mesh,
        out_type=jax.ShapeDtypeStruct((n_tokens, _D), jnp.float32),
        scratch_types=[
            pltpu.VMEM((b_per_w,), jnp.int32),
            pltpu.VMEM((b_per_w, _D), jnp.float32),
            pltpu.SemaphoreType.DMA,
        ],
    )
    def gather_k(cb_hbm, idx_hbm, out_hbm, idx_v, rows_v, sem):
        wid = lax.axis_index("s") * _SC_CORES + lax.axis_index("c")
        base = wid * b_per_w
        pltpu.sync_copy(idx_hbm.at[pl.ds(base, b_per_w)], idx_v)
        pltpu.async_copy(cb_hbm.at[idx_v], rows_v, sem).wait()
        pltpu.sync_copy(rows_v, out_hbm.at[pl.ds(base, b_per_w)])

    return gather_k(codebook, idx_flat)


def kernel(z, codebook):
    B, C, H, W = z.shape
    n_tokens = B * H * W
    idx3, losses = _argmin_call(z.reshape(B, C, H * W), codebook)
    idx_flat = idx3.reshape(n_tokens)
    qf = _sc_gather(codebook, idx_flat, n_tokens)
    quantized = jnp.transpose(qf.reshape(B, H, W, C), (0, 3, 1, 2))
    quantized_st = z + (quantized - z)
    return (quantized_st,
            idx_flat.reshape(B, H, W),
            losses[0, 1],
            losses[0, 0])
